# baseline (speedup 1.0000x reference)
import functools

import jax
import jax.numpy as jnp
from jax.experimental import pallas as pl


def _gcn_body(x_ref, a_ref, w1_ref, b1_ref, w2_ref, b2_ref, o_ref):
    slope = jnp.float32(0.01)
    s = jnp.dot(a_ref[...], x_ref[...], preferred_element_type=jnp.float32)
    h = jnp.dot(s, w1_ref[...], preferred_element_type=jnp.float32) + b1_ref[...]
    h = jnp.where(h >= 0, h, slope * h)
    s2 = jnp.dot(a_ref[...], h, preferred_element_type=jnp.float32)
    o = jnp.dot(s2, w2_ref[...], preferred_element_type=jnp.float32) + b2_ref[...]
    o_ref[...] = jnp.where(o >= 0, o, slope * o)


@functools.partial(jax.jit, static_argnames=("grp",))
def _gcn_block(Xr, A, W1e, b1t, W2e, b2t, grp):
    N = A.shape[0]
    kin_blk = W1e.shape[0]
    kout_blk = W2e.shape[1]
    steps = Xr.shape[1] // kin_blk
    return pl.pallas_call(
        _gcn_body,
        grid=(steps,),
        in_specs=[
            pl.BlockSpec((N, kin_blk), lambda g: (0, g)),
            pl.BlockSpec((N, N), lambda g: (0, 0)),
            pl.BlockSpec((kin_blk, W1e.shape[1]), lambda g: (0, 0)),
            pl.BlockSpec((1, W1e.shape[1]), lambda g: (0, 0)),
            pl.BlockSpec((W2e.shape[0], kout_blk), lambda g: (0, 0)),
            pl.BlockSpec((1, kout_blk), lambda g: (0, 0)),
        ],
        out_specs=pl.BlockSpec((N, kout_blk), lambda g: (0, g)),
        out_shape=jax.ShapeDtypeStruct((N, steps * kout_blk), jnp.float32),
    )(Xr, A, W1e, b1t, W2e, b2t)


def kernel(X, A, W1, b1, W2, b2):
    B, N, T, F_in = X.shape
    F_sp = W1.shape[1]
    BT = B * T
    grp = 8
    assert BT % grp == 0

    Xr = jnp.transpose(X, (1, 0, 2, 3)).reshape(N, BT * F_in)

    eye = jnp.eye(grp, dtype=jnp.float32)
    W1e = jnp.kron(eye, W1)
    W2e = jnp.kron(eye, W2)
    b1t = jnp.tile(b1, grp)[None, :]
    b2t = jnp.tile(b2, grp)[None, :]

    out = _gcn_block(Xr, A, W1e, b1t, W2e, b2t, grp)
    return out.reshape(N, B, T, F_sp).transpose(1, 0, 2, 3)


# speedup vs baseline: 1.3438x; 1.3438x over previous
import functools

import jax
import jax.numpy as jnp
from jax.experimental import pallas as pl


def _gcn_body(x_ref, a_ref, w1_ref, b1_ref, w2_ref, b2_ref, o_ref):
    slope = jnp.float32(0.01)
    s = jnp.dot(a_ref[...], x_ref[...], preferred_element_type=jnp.float32)
    h = jnp.dot(s, w1_ref[...], preferred_element_type=jnp.float32) + b1_ref[...]
    h = jnp.where(h >= 0, h, slope * h)
    s2 = jnp.dot(a_ref[...], h, preferred_element_type=jnp.float32)
    o = jnp.dot(s2, w2_ref[...], preferred_element_type=jnp.float32) + b2_ref[...]
    o_ref[...] = jnp.where(o >= 0, o, slope * o)


@functools.partial(jax.jit, static_argnames=("grp",))
def _gcn_block(Xr, A, W1e, b1t, W2e, b2t, grp):
    N = A.shape[0]
    kin_blk = W1e.shape[0]
    kout_blk = W2e.shape[1]
    steps = Xr.shape[1] // kin_blk
    return pl.pallas_call(
        _gcn_body,
        grid=(steps,),
        in_specs=[
            pl.BlockSpec((N, kin_blk), lambda g: (0, g)),
            pl.BlockSpec((N, N), lambda g: (0, 0)),
            pl.BlockSpec((kin_blk, W1e.shape[1]), lambda g: (0, 0)),
            pl.BlockSpec((1, W1e.shape[1]), lambda g: (0, 0)),
            pl.BlockSpec((W2e.shape[0], kout_blk), lambda g: (0, 0)),
            pl.BlockSpec((1, kout_blk), lambda g: (0, 0)),
        ],
        out_specs=pl.BlockSpec((N, kout_blk), lambda g: (0, g)),
        out_shape=jax.ShapeDtypeStruct((N, steps * kout_blk), jnp.float32),
    )(Xr, A, W1e, b1t, W2e, b2t)


def kernel(X, A, W1, b1, W2, b2):
    B, N, T, F_in = X.shape
    F_sp = W1.shape[1]
    BT = B * T
    grp = 24
    assert BT % grp == 0

    Xr = jnp.transpose(X, (1, 0, 2, 3)).reshape(N, BT * F_in)

    eye = jnp.eye(grp, dtype=jnp.float32)
    W1e = jnp.kron(eye, W1)
    W2e = jnp.kron(eye, W2)
    b1t = jnp.tile(b1, grp)[None, :]
    b2t = jnp.tile(b2, grp)[None, :]

    out = _gcn_block(Xr, A, W1e, b1t, W2e, b2t, grp)
    return out.reshape(N, B, T, F_sp).transpose(1, 0, 2, 3)
